# Initial kernel scaffold; baseline (speedup 1.0000x reference)
import jax
import jax.numpy as jnp
from jax.experimental import pallas as pl


def kernel(x, edge_index, edge_weight, y_target, W1, b1, W2, b2, W3, b3, Wm, bm, gm, bem, Wv, bv, gv, bev):
    raise NotImplementedError("write your pallas kernel here")


# speedup vs baseline: 3.7180x; 3.7180x over previous
import functools

import jax
import jax.numpy as jnp
from jax import lax
from jax.experimental import pallas as pl
from jax.experimental.pallas import tpu as pltpu
from jax.experimental.pallas import tpu_sc as plsc

_N = 10000
_E = 160000
_B = 2
_NC, _NS, _L = 2, 16, 16
_NW = _NC * _NS
_EPAD = ((_E + _NW * _L - 1) // (_NW * _L)) * (_NW * _L)
_EPT = _EPAD // _NW
_T = _EPT // _L
_C = 128
_NPAD = 10240
_RS = _NPAD // _NS
_BM = 1000
_NBLK = (_B * _N) // _BM

_mesh = plsc.VectorSubcoreMesh(core_axis_name="c", subcore_axis_name="s")
_sc_params = pltpu.CompilerParams(needs_layout_passes=False)


def _sc_deg_body(dst_h, ew_h, zeros_h, out_h, dst_v, ew_v, acc, sem):
    c = lax.axis_index("c")
    s = lax.axis_index("s")
    wid = s * _NC + c
    base = pl.multiple_of(wid * _EPT, 8)
    pltpu.sync_copy(dst_h.at[pl.ds(base, _EPT)], dst_v)
    pltpu.sync_copy(ew_h.at[pl.ds(base, _EPT)], ew_v)

    off = pl.multiple_of(s * _RS, 8)
    pltpu.sync_copy(zeros_h, acc.at[pl.ds(off, _RS)])
    plsc.subcore_barrier()

    def step(t, carry):
        e0 = t * _L
        dvec = dst_v[pl.ds(e0, _L)]
        pltpu.sync_copy(ew_v.at[pl.ds(e0, _L)], acc.at[dvec], add=True)
        return carry

    lax.fori_loop(0, _T, step, 0)
    plsc.subcore_barrier()
    oout = pl.multiple_of(c * _NPAD + s * _RS, 8)
    pltpu.sync_copy(acc.at[pl.ds(off, _RS)], out_h.at[pl.ds(oout, _RS)])


_sc_deg = pl.kernel(
    _sc_deg_body,
    out_type=jax.ShapeDtypeStruct((_NC * _NPAD,), jnp.float32),
    mesh=_mesh,
    compiler_params=_sc_params,
    scratch_types=[
        pltpu.VMEM((_EPT,), jnp.int32),
        pltpu.VMEM((_EPT,), jnp.float32),
        pltpu.VMEM_SHARED((_NPAD,), jnp.float32),
        pltpu.SemaphoreType.DMA,
    ],
)


def _sc_prop_body(nch, tbl_h, src_h, dst_h, ew_h, zeros_h, out_h,
                  src_v, dst_v, ew_v, rows, acc, sem):
    c = lax.axis_index("c")
    s = lax.axis_index("s")
    wid = s * _NC + c
    base = pl.multiple_of(wid * _EPT, 8)
    pltpu.sync_copy(src_h.at[pl.ds(base, _EPT)], src_v)
    pltpu.sync_copy(dst_h.at[pl.ds(base, _EPT)], dst_v)
    pltpu.sync_copy(ew_h.at[pl.ds(base, _EPT)], ew_v)

    for ch in range(nch):
        roff = pl.multiple_of(s * _RS, 8)
        pltpu.sync_copy(zeros_h, acc.at[pl.ds(roff, _RS)])
        plsc.subcore_barrier()

        def step(t, carry):
            e0 = t * _L
            svec = src_v[pl.ds(e0, _L)]
            pltpu.async_copy(tbl_h.at[ch].at[svec], rows, sem).wait()
            for j in range(_L):
                wv = plsc.load_gather(ew_v, [jnp.full((_L,), e0 + j, jnp.int32)])
                for cc in range(_C // _L):
                    sl = pl.ds(cc * _L, _L)
                    rows[j, sl] = rows[j, sl] * wv
            dvec = dst_v[pl.ds(e0, _L)]
            pltpu.sync_copy(rows, acc.at[dvec], add=True)
            return carry

        lax.fori_loop(0, _T, step, 0)
        plsc.subcore_barrier()
        pltpu.sync_copy(acc.at[pl.ds(roff, _RS)],
                        out_h.at[c, ch, pl.ds(roff, _RS)])


def _mk_sc_prop(nch):
    return pl.kernel(
        functools.partial(_sc_prop_body, nch),
        out_type=jax.ShapeDtypeStruct((_NC, nch, _NPAD, _C), jnp.float32),
        mesh=_mesh,
        compiler_params=_sc_params,
        scratch_types=[
            pltpu.VMEM((_EPT,), jnp.int32),
            pltpu.VMEM((_EPT,), jnp.int32),
            pltpu.VMEM((_EPT,), jnp.float32),
            pltpu.VMEM((_L, _C), jnp.float32),
            pltpu.VMEM_SHARED((_NPAD, _C), jnp.float32),
            pltpu.SemaphoreType.DMA,
        ],
    )


_sc_prop = {n: _mk_sc_prop(n) for n in (2, 8, 4)}


def _l2n_relu(h):
    n = jnp.sqrt(jnp.sum(h * h, axis=-1, keepdims=True))
    return jnp.maximum(h / jnp.maximum(n, 1e-12), 0.0)


def _mm1_body(S_ref, xs_ref, dinv_ref, W_ref, b_ref, o_ref):
    dinv = dinv_ref[...]
    xp = dinv * (S_ref[...] + dinv * xs_ref[...])
    h = jnp.dot(xp, W_ref[...], preferred_element_type=jnp.float32) + b_ref[...]
    o_ref[...] = _l2n_relu(h)


def _mm2_body(h_ref, W_ref, dinv_ref, o_ref):
    t = jnp.dot(h_ref[...], W_ref[...], preferred_element_type=jnp.float32)
    o_ref[...] = dinv_ref[...] * t


def _mm3_body(S_ref, ts_ref, dinv_ref, b_ref, W_ref, o_ref):
    dinv = dinv_ref[...]
    h2 = _l2n_relu(dinv * (S_ref[...] + ts_ref[...]) + b_ref[...])
    t = jnp.dot(h2, W_ref[...], preferred_element_type=jnp.float32)
    o_ref[...] = dinv * t


def _final_body(S_ref, ts_ref, dinv_ref, b_ref, o_ref):
    dinv = dinv_ref[...]
    y = _l2n_relu(dinv * (S_ref[...] + ts_ref[...]) + b_ref[...])
    par = lax.broadcasted_iota(jnp.int32, (_BM, _B), 0) % _B
    col = lax.broadcasted_iota(jnp.int32, (_BM, _B), 1)
    sel = (par == col).astype(jnp.float32)
    psum = lax.dot_general(sel, y, (((0,), (0,)), ((), ())),
                           preferred_element_type=jnp.float32)
    o_ref[...] = psum.reshape(1, _B, y.shape[-1])


def _head_body(p_ref, yt_ref, Wm_ref, bm_ref, gm_ref, bem_ref,
               Wv_ref, bv_ref, gv_ref, bev_ref, mu_ref, lv_ref):
    ge = jnp.sum(p_ref[...], axis=0)
    yt = yt_ref[...]

    def branch(W, b, g, be):
        u = lax.dot_general(ge, W[:256, :], (((1,), (0,)), ((), ())),
                            preferred_element_type=jnp.float32)
        u = u + yt * W[256:257, :] + b
        m = jnp.mean(u, axis=0, keepdims=True)
        v = jnp.mean((u - m) * (u - m), axis=0, keepdims=True)
        return jnp.maximum(g * (u - m) / jnp.sqrt(v + 1e-5) + be, 0.0)

    mu_ref[...] = branch(Wm_ref[...], bm_ref[...], gm_ref[...], bem_ref[...])
    t = branch(Wv_ref[...], bv_ref[...], gv_ref[...], bev_ref[...])
    lv_ref[...] = 1.0 / (1.0 + jnp.exp(-t))


def _row_spec(w):
    return pl.BlockSpec((_BM, w), lambda i: (i, 0))


def _full_spec(shape):
    return pl.BlockSpec(shape, lambda i: tuple(0 for _ in shape))


_mm1 = pl.pallas_call(
    _mm1_body, grid=(_NBLK,),
    in_specs=[_row_spec(128), _row_spec(128), _row_spec(1),
              _full_spec((128, 1024)), _full_spec((1, 1024))],
    out_specs=_row_spec(1024),
    out_shape=jax.ShapeDtypeStruct((_B * _N, 1024), jnp.float32),
)

_mm2 = pl.pallas_call(
    _mm2_body, grid=(_NBLK,),
    in_specs=[_row_spec(1024), _full_spec((1024, 512)), _row_spec(1)],
    out_specs=_row_spec(512),
    out_shape=jax.ShapeDtypeStruct((_B * _N, 512), jnp.float32),
)

_mm3 = pl.pallas_call(
    _mm3_body, grid=(_NBLK,),
    in_specs=[_row_spec(512), _row_spec(512), _row_spec(1),
              _full_spec((1, 512)), _full_spec((512, 256))],
    out_specs=_row_spec(256),
    out_shape=jax.ShapeDtypeStruct((_B * _N, 256), jnp.float32),
)

_final = pl.pallas_call(
    _final_body, grid=(_NBLK,),
    in_specs=[_row_spec(256), _row_spec(256), _row_spec(1),
              _full_spec((1, 256))],
    out_specs=pl.BlockSpec((1, _B, 256), lambda i: (i, 0, 0)),
    out_shape=jax.ShapeDtypeStruct((_NBLK, _B, 256), jnp.float32),
)

_head = pl.pallas_call(
    _head_body,
    out_shape=[jax.ShapeDtypeStruct((_B, 128), jnp.float32),
               jax.ShapeDtypeStruct((_B, 128), jnp.float32)],
)


def kernel(x, edge_index, edge_weight, y_target,
           W1, b1, W2, b2, W3, b3, Wm, bm, gm, bem, Wv, bv, gv, bev):
    src = edge_index[0]
    dst = edge_index[1]
    pad = _EPAD - _E
    srcp = jnp.concatenate([src, jnp.zeros((pad,), jnp.int32)])
    dstp = jnp.concatenate([dst, jnp.zeros((pad,), jnp.int32)])
    ewp = jnp.concatenate([edge_weight, jnp.zeros((pad,), jnp.float32)])
    zeros1 = jnp.zeros((_RS,), jnp.float32)
    zeros2 = jnp.zeros((_RS, _C), jnp.float32)

    degp = _sc_deg(dstp, ewp, zeros1)
    deg = 1.0 + degp[:_N] + degp[_NPAD:_NPAD + _N]
    dinv = lax.rsqrt(jnp.maximum(deg, 1e-12))
    dinv2 = jnp.repeat(dinv, _B)[:, None]

    def prop(tbl_node, nch):
        tbl = tbl_node.reshape(_N, nch, _C).transpose(1, 0, 2)
        S = _sc_prop[nch](tbl, srcp, dstp, ewp, zeros2)
        Sn = (S[0] + S[1])[:, :_N, :]
        return Sn.transpose(1, 0, 2).reshape(_B * _N, nch * _C // _B)

    xs = jnp.transpose(x, (1, 0, 2)).reshape(_N, _B * 128)
    S1 = prop(dinv[:, None] * xs, 2)
    h1 = _mm1(S1, xs.reshape(_B * _N, 128), dinv2, W1, b1.reshape(1, -1))
    t2s = _mm2(h1, W2, dinv2)
    S2 = prop(t2s.reshape(_N, _B * 512), 8)
    t3s = _mm3(S2, t2s, dinv2, b2.reshape(1, -1), W3)
    S3 = prop(t3s.reshape(_N, _B * 256), 4)
    parts = _final(S3, t3s, dinv2, b3.reshape(1, -1))
    z_mu, z_lv = _head(parts, y_target, Wm, bm.reshape(1, -1),
                       gm.reshape(1, -1), bem.reshape(1, -1), Wv,
                       bv.reshape(1, -1), gv.reshape(1, -1), bev.reshape(1, -1))
    return (z_mu, z_lv)
